# baseline (speedup 1.0000x reference)
import functools

import jax
import jax.numpy as jnp
from jax import lax
from jax.experimental import pallas as pl
from jax.experimental.pallas import tpu as pltpu
from jax.experimental.pallas import tpu_sc as plsc

N_NODES = 10000
D = 128
NUM_GRAPHS = 64
NC, NS = 2, 16
CH = 128
NCH = 80
EP = NC * NS * NCH * CH
TCH = EP // CH
NCH_P = TCH // NS
DH = D // NC
NP = 10112
GARBAGE = N_NODES
RPT = NP // NS


def _sc_mesh():
    return plsc.VectorSubcoreMesh(core_axis_name="c", subcore_axis_name="s")


def _sc_degree(dst_idx, zeros16, ones16):
    @functools.partial(
        pl.kernel,
        out_type=jax.ShapeDtypeStruct((NC * NP, 16), jnp.float32),
        mesh=_sc_mesh(),
        scratch_types=[
            pltpu.VMEM((NCH, CH), jnp.int32),
            pltpu.VMEM((CH, 16), jnp.float32),
            pltpu.VMEM_SHARED((NP, 16), jnp.float32),
        ],
        compiler_params=pltpu.CompilerParams(use_tc_tiling_on_sc=False),
    )
    def k(dst_hbm, z_hbm, one_hbm, out_hbm, dst_v, ones_v, acc):
        c = lax.axis_index("c")
        s = lax.axis_index("s")
        rows = pl.ds(s * RPT, RPT)
        pltpu.sync_copy(z_hbm.at[rows], acc.at[rows])
        pltpu.sync_copy(dst_hbm.at[c, s], dst_v)
        pltpu.sync_copy(one_hbm, ones_v)
        plsc.subcore_barrier()

        @pl.loop(0, NCH)
        def _(j):
            pltpu.sync_copy(ones_v, acc.at[dst_v.at[j]], add=True)

        plsc.subcore_barrier()
        pltpu.sync_copy(acc.at[rows], out_hbm.at[pl.ds(c * NP + s * RPT, RPT)])

    return k(dst_idx, zeros16, ones16)


def _sc_propagate(src_idx, dst_idx, tables, zeros64):
    @functools.partial(
        pl.kernel,
        out_type=jax.ShapeDtypeStruct((NC * NP, DH), jnp.float32),
        mesh=_sc_mesh(),
        scratch_types=[
            pltpu.VMEM((NCH_P, CH), jnp.int32),
            pltpu.VMEM((NCH_P, CH), jnp.int32),
            pltpu.VMEM((CH, DH), jnp.float32),
            pltpu.VMEM((CH, DH), jnp.float32),
            pltpu.VMEM((CH, DH), jnp.float32),
            pltpu.VMEM((CH, DH), jnp.float32),
            pltpu.VMEM_SHARED((NP, DH), jnp.float32),
            pltpu.SemaphoreType.DMA,
            pltpu.SemaphoreType.DMA,
            pltpu.SemaphoreType.DMA,
            pltpu.SemaphoreType.DMA,
            pltpu.SemaphoreType.DMA,
            pltpu.SemaphoreType.DMA,
            pltpu.SemaphoreType.DMA,
            pltpu.SemaphoreType.DMA,
        ],
        compiler_params=pltpu.CompilerParams(use_tc_tiling_on_sc=False),
    )
    def k(src_hbm, dst_hbm, tab_hbm, z_hbm, out_hbm,
          src_v, dst_v, b0, b1, b2, b3, acc,
          gs0, gs1, gs2, gs3, ss0, ss1, ss2, ss3):
        c = lax.axis_index("c")
        s = lax.axis_index("s")
        rows = pl.ds(s * RPT, RPT)
        pltpu.sync_copy(z_hbm.at[rows], acc.at[rows])
        pltpu.sync_copy(src_hbm.at[s], src_v)
        pltpu.sync_copy(dst_hbm.at[s], dst_v)
        plsc.subcore_barrier()

        B = (b0, b1, b2, b3)
        GS = (gs0, gs1, gs2, gs3)
        SS = (ss0, ss1, ss2, ss3)
        tab_c = tab_hbm.at[c]

        def g_start(i, slot):
            pltpu.make_async_copy(tab_c.at[src_v.at[i]], B[slot], GS[slot]).start()

        def g_wait(i, slot):
            pltpu.make_async_copy(tab_c.at[src_v.at[i]], B[slot], GS[slot]).wait()

        def s_start(i, slot):
            pltpu.async_copy(B[slot], acc.at[dst_v.at[i]], SS[slot], add=True)

        def s_wait(i, slot):
            pltpu.make_async_copy(B[slot], acc.at[dst_v.at[i]], SS[slot]).wait()

        g_start(0, 0)
        g_start(1, 1)
        g_wait(0, 0); s_start(0, 0); g_start(2, 2)
        g_wait(1, 1); s_start(1, 1); g_start(3, 3)

        @pl.loop(2, NCH_P - 2, step=4)
        def _(j):
            for k_ in range(4):
                i = j + k_
                slot = (2 + k_) % 4
                g_wait(i, slot)
                s_start(i, slot)
                s_wait(i - 2, (slot + 2) % 4)
                g_start(i + 2, (slot + 2) % 4)

        g_wait(NCH_P - 2, 2); s_start(NCH_P - 2, 2); s_wait(NCH_P - 4, 0)
        g_wait(NCH_P - 1, 3); s_start(NCH_P - 1, 3); s_wait(NCH_P - 3, 1)
        s_wait(NCH_P - 2, 2)
        s_wait(NCH_P - 1, 3)

        plsc.subcore_barrier()
        pltpu.sync_copy(acc.at[rows], out_hbm.at[pl.ds(c * NP + s * RPT, RPT)])

    return k(src_idx, dst_idx, tables, zeros64)


def _tc_matmul(xp, W1):
    def body(x_ref, w_ref, o_ref):
        o_ref[...] = jnp.dot(x_ref[...], w_ref[...],
                             preferred_element_type=jnp.float32)

    return pl.pallas_call(
        body, out_shape=jax.ShapeDtypeStruct((NP, D), jnp.float32))(xp, W1)


def _tc_scale(h1, degs):
    def body(h_ref, d_ref, g_ref, s_ref):
        deg = d_ref[:NP, 0:1] + d_ref[NP:, 0:1] + 1.0
        dinv = lax.rsqrt(deg)
        g = h_ref[...] * dinv
        g_ref[:NP, :] = g[:, :DH]
        g_ref[NP:, :] = g[:, DH:]
        s_ref[...] = jnp.broadcast_to(dinv, (NP, D))

    return pl.pallas_call(
        body,
        out_shape=[jax.ShapeDtypeStruct((NC * NP, DH), jnp.float32),
                   jax.ShapeDtypeStruct((NP, D), jnp.float32)],
    )(h1, degs)


def _tc_layer(Pcols, g1cols, dinvb, b1, W2):
    def body(p_ref, g_ref, s_ref, b_ref, w_ref, o_ref):
        dinv = s_ref[...]
        acc = jnp.concatenate(
            [p_ref[:NP, :] + g_ref[:NP, :], p_ref[NP:, :] + g_ref[NP:, :]],
            axis=1)
        h = jnp.maximum(acc * dinv + b_ref[...], 0.0)
        g2 = jnp.dot(h, w_ref[...],
                     preferred_element_type=jnp.float32) * dinv
        o_ref[:NP, :] = g2[:, :DH]
        o_ref[NP:, :] = g2[:, DH:]

    return pl.pallas_call(
        body, out_shape=jax.ShapeDtypeStruct((NC * NP, DH), jnp.float32),
    )(Pcols, g1cols, dinvb, b1, W2)


def _tc_head(Qcols, g2cols, dinvb, b2, batch_p, fc_w, fc_b):
    def body(q_ref, g_ref, s_ref, b_ref, bt_ref, w_ref, c_ref, o_ref):
        acc = jnp.concatenate(
            [q_ref[:NP, :] + g_ref[:NP, :], q_ref[NP:, :] + g_ref[NP:, :]],
            axis=1)
        h = jnp.maximum(acc * s_ref[...] + b_ref[...], 0.0)
        gid = bt_ref[...]
        oh = (gid[:, None] == lax.broadcasted_iota(
            jnp.int32, (NP, NUM_GRAPHS), 1)).astype(jnp.float32)
        sums = lax.dot_general(oh, h, (((0,), (0,)), ((), ())),
                               preferred_element_type=jnp.float32)
        counts = jnp.sum(oh, axis=0)
        pooled = sums / jnp.maximum(counts, 1.0)[:, None]
        o_ref[...] = jnp.dot(pooled, w_ref[...],
                             preferred_element_type=jnp.float32) + c_ref[...]

    return pl.pallas_call(
        body,
        out_shape=jax.ShapeDtypeStruct((NUM_GRAPHS, fc_w.shape[1]), jnp.float32),
    )(Qcols, g2cols, dinvb, b2, batch_p, fc_w, fc_b)


def kernel(x, edge_index, batch, W1, b1, W2, b2, fc_w, fc_b):
    i32 = jnp.int32
    n = x.shape[0]
    src = edge_index[0].astype(i32)
    dst = edge_index[1].astype(i32)
    pad_e = EP - src.shape[0]
    src_f = jnp.concatenate([src, jnp.zeros((pad_e,), i32)])
    dst_f = jnp.concatenate([dst, jnp.full((pad_e,), GARBAGE, i32)])
    dst_deg = dst_f.reshape(NC, NS, NCH, CH)
    src_prop = src_f.reshape(NS, NCH_P, CH)
    dst_prop = dst_f.reshape(NS, NCH_P, CH)
    xp = jnp.pad(x, ((0, NP - n), (0, 0)))
    batch_p = jnp.concatenate([batch.astype(i32), jnp.full((NP - n,), NUM_GRAPHS, i32)])
    zeros16 = jnp.zeros((NP, 16), jnp.float32)
    ones16 = jnp.ones((CH, 16), jnp.float32)
    zeros64 = jnp.zeros((NP, DH), jnp.float32)

    degs = _sc_degree(dst_deg, zeros16, ones16)
    h1 = _tc_matmul(xp, W1)
    g1cols, dinvb = _tc_scale(h1, degs)
    P = _sc_propagate(src_prop, dst_prop,
                      g1cols.reshape(NC, NP, DH), zeros64)
    g2cols = _tc_layer(P, g1cols, dinvb, b1, W2)
    Q = _sc_propagate(src_prop, dst_prop,
                      g2cols.reshape(NC, NP, DH), zeros64)
    return _tc_head(Q, g2cols, dinvb, b2, batch_p, fc_w, fc_b)


# speedup vs baseline: 1.0911x; 1.0911x over previous
import functools

import jax
import jax.numpy as jnp
from jax import lax
from jax.experimental import pallas as pl
from jax.experimental.pallas import tpu as pltpu
from jax.experimental.pallas import tpu_sc as plsc

N_NODES = 10000
D = 128
NUM_GRAPHS = 64
NC, NS = 2, 16
CH = 128
NCH = 80
EP = NC * NS * NCH * CH
TCH = EP // CH
NCH_P = TCH // NS
DH = D // NC
NP = 10112
GARBAGE = N_NODES
RPT = NP // NS


def _sc_mesh():
    return plsc.VectorSubcoreMesh(core_axis_name="c", subcore_axis_name="s")


def _sc_degree(dst_idx, zeros16, ones16):
    @functools.partial(
        pl.kernel,
        out_type=jax.ShapeDtypeStruct((NC * NP, 16), jnp.float32),
        mesh=_sc_mesh(),
        scratch_types=[
            pltpu.VMEM((NCH, CH), jnp.int32),
            pltpu.VMEM((CH, 16), jnp.float32),
            pltpu.VMEM_SHARED((NP, 16), jnp.float32),
        ],
        compiler_params=pltpu.CompilerParams(use_tc_tiling_on_sc=False),
    )
    def k(dst_hbm, z_hbm, one_hbm, out_hbm, dst_v, ones_v, acc):
        c = lax.axis_index("c")
        s = lax.axis_index("s")
        rows = pl.ds(s * RPT, RPT)
        pltpu.sync_copy(z_hbm.at[rows], acc.at[rows])
        pltpu.sync_copy(dst_hbm.at[c, s], dst_v)
        pltpu.sync_copy(one_hbm, ones_v)
        plsc.subcore_barrier()

        @pl.loop(0, NCH)
        def _(j):
            pltpu.sync_copy(ones_v, acc.at[dst_v.at[j]], add=True)

        plsc.subcore_barrier()
        pltpu.sync_copy(acc.at[rows], out_hbm.at[pl.ds(c * NP + s * RPT, RPT)])

    return k(dst_idx, zeros16, ones16)


def _sc_propagate(src_idx, dst_idx, tables, zeros64):
    @functools.partial(
        pl.kernel,
        out_type=jax.ShapeDtypeStruct((NC * NP, DH), jnp.float32),
        mesh=_sc_mesh(),
        scratch_types=[
            pltpu.VMEM((NCH_P, CH), jnp.int32),
            pltpu.VMEM((NCH_P, CH), jnp.int32),
            pltpu.VMEM((CH, DH), jnp.float32),
            pltpu.VMEM((CH, DH), jnp.float32),
            pltpu.VMEM((CH, DH), jnp.float32),
            pltpu.VMEM((CH, DH), jnp.float32),
            pltpu.VMEM_SHARED((NP, DH), jnp.float32),
            pltpu.SemaphoreType.DMA,
            pltpu.SemaphoreType.DMA,
            pltpu.SemaphoreType.DMA,
            pltpu.SemaphoreType.DMA,
            pltpu.SemaphoreType.DMA,
            pltpu.SemaphoreType.DMA,
            pltpu.SemaphoreType.DMA,
            pltpu.SemaphoreType.DMA,
        ],
        compiler_params=pltpu.CompilerParams(use_tc_tiling_on_sc=False),
    )
    def k(src_hbm, dst_hbm, tab_hbm, z_hbm, out_hbm,
          src_v, dst_v, b0, b1, b2, b3, acc,
          gs0, gs1, gs2, gs3, ss0, ss1, ss2, ss3):
        c = lax.axis_index("c")
        s = lax.axis_index("s")
        rows = pl.ds(s * RPT, RPT)
        pltpu.sync_copy(z_hbm.at[rows], acc.at[rows])
        pltpu.sync_copy(src_hbm.at[c, s], src_v)
        pltpu.sync_copy(dst_hbm.at[s], dst_v)
        plsc.subcore_barrier()

        B = (b0, b1, b2, b3)
        GS = (gs0, gs1, gs2, gs3)
        SS = (ss0, ss1, ss2, ss3)

        def g_start(i, slot):
            pltpu.make_async_copy(tab_hbm.at[src_v.at[i]], B[slot], GS[slot]).start()

        def g_wait(i, slot):
            pltpu.make_async_copy(tab_hbm.at[src_v.at[i]], B[slot], GS[slot]).wait()

        def s_start(i, slot):
            pltpu.async_copy(B[slot], acc.at[dst_v.at[i]], SS[slot], add=True)

        def s_wait(i, slot):
            pltpu.make_async_copy(B[slot], acc.at[dst_v.at[i]], SS[slot]).wait()

        g_start(0, 0)
        g_start(1, 1)
        g_wait(0, 0); s_start(0, 0); g_start(2, 2)
        g_wait(1, 1); s_start(1, 1); g_start(3, 3)

        @pl.loop(2, NCH_P - 2, step=4)
        def _(j):
            for k_ in range(4):
                i = j + k_
                slot = (2 + k_) % 4
                g_wait(i, slot)
                s_start(i, slot)
                s_wait(i - 2, (slot + 2) % 4)
                g_start(i + 2, (slot + 2) % 4)

        g_wait(NCH_P - 2, 2); s_start(NCH_P - 2, 2); s_wait(NCH_P - 4, 0)
        g_wait(NCH_P - 1, 3); s_start(NCH_P - 1, 3); s_wait(NCH_P - 3, 1)
        s_wait(NCH_P - 2, 2)
        s_wait(NCH_P - 1, 3)

        plsc.subcore_barrier()
        pltpu.sync_copy(acc.at[rows], out_hbm.at[pl.ds(c * NP + s * RPT, RPT)])

    return k(src_idx, dst_idx, tables, zeros64)


def _tc_matmul(xp, W1):
    def body(x_ref, w_ref, o_ref):
        o_ref[...] = jnp.dot(x_ref[...], w_ref[...],
                             preferred_element_type=jnp.float32)

    return pl.pallas_call(
        body, out_shape=jax.ShapeDtypeStruct((NP, D), jnp.float32))(xp, W1)


def _tc_scale(h1, degs):
    def body(h_ref, d_ref, g_ref, s_ref):
        deg = d_ref[:NP, 0:1] + d_ref[NP:, 0:1] + 1.0
        dinv = lax.rsqrt(deg)
        g = h_ref[...] * dinv
        g_ref[:NP, :] = g[:, :DH]
        g_ref[NP:, :] = g[:, DH:]
        s_ref[...] = jnp.broadcast_to(dinv, (NP, D))

    return pl.pallas_call(
        body,
        out_shape=[jax.ShapeDtypeStruct((NC * NP, DH), jnp.float32),
                   jax.ShapeDtypeStruct((NP, D), jnp.float32)],
    )(h1, degs)


def _tc_layer(Pcols, g1cols, dinvb, b1, W2):
    def body(p_ref, g_ref, s_ref, b_ref, w_ref, o_ref):
        dinv = s_ref[...]
        acc = jnp.concatenate(
            [p_ref[:NP, :] + g_ref[:NP, :], p_ref[NP:, :] + g_ref[NP:, :]],
            axis=1)
        h = jnp.maximum(acc * dinv + b_ref[...], 0.0)
        g2 = jnp.dot(h, w_ref[...],
                     preferred_element_type=jnp.float32) * dinv
        o_ref[:NP, :] = g2[:, :DH]
        o_ref[NP:, :] = g2[:, DH:]

    return pl.pallas_call(
        body, out_shape=jax.ShapeDtypeStruct((NC * NP, DH), jnp.float32),
    )(Pcols, g1cols, dinvb, b1, W2)


def _tc_head(Qcols, g2cols, dinvb, b2, batch_p, fc_w, fc_b):
    def body(q_ref, g_ref, s_ref, b_ref, bt_ref, w_ref, c_ref, o_ref):
        acc = jnp.concatenate(
            [q_ref[:NP, :] + g_ref[:NP, :], q_ref[NP:, :] + g_ref[NP:, :]],
            axis=1)
        h = jnp.maximum(acc * s_ref[...] + b_ref[...], 0.0)
        gid = bt_ref[...]
        oh = (gid[:, None] == lax.broadcasted_iota(
            jnp.int32, (NP, NUM_GRAPHS), 1)).astype(jnp.float32)
        sums = lax.dot_general(oh, h, (((0,), (0,)), ((), ())),
                               preferred_element_type=jnp.float32)
        counts = jnp.sum(oh, axis=0)
        pooled = sums / jnp.maximum(counts, 1.0)[:, None]
        o_ref[...] = jnp.dot(pooled, w_ref[...],
                             preferred_element_type=jnp.float32) + c_ref[...]

    return pl.pallas_call(
        body,
        out_shape=jax.ShapeDtypeStruct((NUM_GRAPHS, fc_w.shape[1]), jnp.float32),
    )(Qcols, g2cols, dinvb, b2, batch_p, fc_w, fc_b)


def kernel(x, edge_index, batch, W1, b1, W2, b2, fc_w, fc_b):
    i32 = jnp.int32
    n = x.shape[0]
    src = edge_index[0].astype(i32)
    dst = edge_index[1].astype(i32)
    pad_e = EP - src.shape[0]
    src_f = jnp.concatenate([src, jnp.zeros((pad_e,), i32)])
    dst_f = jnp.concatenate([dst, jnp.full((pad_e,), GARBAGE, i32)])
    dst_deg = dst_f.reshape(NC, NS, NCH, CH)
    src_prop = jnp.stack([src_f, src_f + NP]).reshape(NC, NS, NCH_P, CH)
    dst_prop = dst_f.reshape(NS, NCH_P, CH)
    xp = jnp.pad(x, ((0, NP - n), (0, 0)))
    batch_p = jnp.concatenate([batch.astype(i32), jnp.full((NP - n,), NUM_GRAPHS, i32)])
    zeros16 = jnp.zeros((NP, 16), jnp.float32)
    ones16 = jnp.ones((CH, 16), jnp.float32)
    zeros64 = jnp.zeros((NP, DH), jnp.float32)

    degs = _sc_degree(dst_deg, zeros16, ones16)
    h1 = _tc_matmul(xp, W1)
    g1cols, dinvb = _tc_scale(h1, degs)
    P = _sc_propagate(src_prop, dst_prop, g1cols, zeros64)
    g2cols = _tc_layer(P, g1cols, dinvb, b1, W2)
    Q = _sc_propagate(src_prop, dst_prop, g2cols, zeros64)
    return _tc_head(Q, g2cols, dinvb, b2, batch_p, fc_w, fc_b)


# speedup vs baseline: 1.0985x; 1.0067x over previous
import functools

import jax
import jax.numpy as jnp
from jax import lax
from jax.experimental import pallas as pl
from jax.experimental.pallas import tpu as pltpu
from jax.experimental.pallas import tpu_sc as plsc

N_NODES = 10000
D = 128
NUM_GRAPHS = 64
NC, NS = 2, 16
CH = 128
NCH = 80
EP = NC * NS * NCH * CH
TCH = EP // CH
NCH_P = TCH // NS
DH = D // NC
NP = 10112
GARBAGE = N_NODES
RPT = NP // NS


def _sc_mesh():
    return plsc.VectorSubcoreMesh(core_axis_name="c", subcore_axis_name="s")


def _sc_degree(dst_idx, zeros16, ones16):
    @functools.partial(
        pl.kernel,
        out_type=jax.ShapeDtypeStruct((NC * NP, 16), jnp.float32),
        mesh=_sc_mesh(),
        scratch_types=[
            pltpu.VMEM((NCH, CH), jnp.int32),
            pltpu.VMEM((CH, 16), jnp.float32),
            pltpu.VMEM_SHARED((NP, 16), jnp.float32),
            pltpu.SemaphoreType.DMA,
            pltpu.SemaphoreType.DMA,
            pltpu.SemaphoreType.DMA,
        ],
        compiler_params=pltpu.CompilerParams(use_tc_tiling_on_sc=False),
    )
    def k(dst_hbm, z_hbm, one_hbm, out_hbm, dst_v, ones_v, acc,
          sm0, sm1, sm2):
        c = lax.axis_index("c")
        s = lax.axis_index("s")
        rows = pl.ds(s * RPT, RPT)
        cp0 = pltpu.async_copy(z_hbm.at[rows], acc.at[rows], sm0)
        cp1 = pltpu.async_copy(dst_hbm.at[c, s], dst_v, sm1)
        cp2 = pltpu.async_copy(one_hbm, ones_v, sm2)
        cp0.wait(); cp1.wait(); cp2.wait()
        plsc.subcore_barrier()

        @pl.loop(0, NCH)
        def _(j):
            pltpu.sync_copy(ones_v, acc.at[dst_v.at[j]], add=True)

        plsc.subcore_barrier()
        pltpu.sync_copy(acc.at[rows], out_hbm.at[pl.ds(c * NP + s * RPT, RPT)])

    return k(dst_idx, zeros16, ones16)


def _sc_propagate(src_idx, dst_idx, tables, zeros64):
    @functools.partial(
        pl.kernel,
        out_type=jax.ShapeDtypeStruct((NC * NP, DH), jnp.float32),
        mesh=_sc_mesh(),
        scratch_types=[
            pltpu.VMEM((NCH_P, CH), jnp.int32),
            pltpu.VMEM((NCH_P, CH), jnp.int32),
            pltpu.VMEM((CH, DH), jnp.float32),
            pltpu.VMEM((CH, DH), jnp.float32),
            pltpu.VMEM((CH, DH), jnp.float32),
            pltpu.VMEM((CH, DH), jnp.float32),
            pltpu.VMEM_SHARED((NP, DH), jnp.float32),
            pltpu.SemaphoreType.DMA,
            pltpu.SemaphoreType.DMA,
            pltpu.SemaphoreType.DMA,
            pltpu.SemaphoreType.DMA,
            pltpu.SemaphoreType.DMA,
            pltpu.SemaphoreType.DMA,
            pltpu.SemaphoreType.DMA,
            pltpu.SemaphoreType.DMA,
        ],
        compiler_params=pltpu.CompilerParams(use_tc_tiling_on_sc=False),
    )
    def k(src_hbm, dst_hbm, tab_hbm, z_hbm, out_hbm,
          src_v, dst_v, b0, b1, b2, b3, acc,
          gs0, gs1, gs2, gs3, ss0, ss1, ss2, ss3):
        c = lax.axis_index("c")
        s = lax.axis_index("s")
        rows = pl.ds(s * RPT, RPT)
        cp0 = pltpu.async_copy(z_hbm.at[rows], acc.at[rows], gs0)
        cp1 = pltpu.async_copy(src_hbm.at[c, s], src_v, gs1)
        cp2 = pltpu.async_copy(dst_hbm.at[s], dst_v, gs2)
        cp0.wait(); cp1.wait(); cp2.wait()
        plsc.subcore_barrier()

        B = (b0, b1, b2, b3)
        GS = (gs0, gs1, gs2, gs3)
        SS = (ss0, ss1, ss2, ss3)

        def g_start(i, slot):
            pltpu.make_async_copy(tab_hbm.at[src_v.at[i]], B[slot], GS[slot]).start()

        def g_wait(i, slot):
            pltpu.make_async_copy(tab_hbm.at[src_v.at[i]], B[slot], GS[slot]).wait()

        def s_start(i, slot):
            pltpu.async_copy(B[slot], acc.at[dst_v.at[i]], SS[slot], add=True)

        def s_wait(i, slot):
            pltpu.make_async_copy(B[slot], acc.at[dst_v.at[i]], SS[slot]).wait()

        g_start(0, 0)
        g_start(1, 1)
        g_wait(0, 0); s_start(0, 0); g_start(2, 2)
        g_wait(1, 1); s_start(1, 1); g_start(3, 3)

        @pl.loop(2, NCH_P - 2, step=4)
        def _(j):
            for k_ in range(4):
                i = j + k_
                slot = (2 + k_) % 4
                g_wait(i, slot)
                s_start(i, slot)
                s_wait(i - 2, (slot + 2) % 4)
                g_start(i + 2, (slot + 2) % 4)

        g_wait(NCH_P - 2, 2); s_start(NCH_P - 2, 2); s_wait(NCH_P - 4, 0)
        g_wait(NCH_P - 1, 3); s_start(NCH_P - 1, 3); s_wait(NCH_P - 3, 1)
        s_wait(NCH_P - 2, 2)
        s_wait(NCH_P - 1, 3)

        plsc.subcore_barrier()
        pltpu.sync_copy(acc.at[rows], out_hbm.at[pl.ds(c * NP + s * RPT, RPT)])

    return k(src_idx, dst_idx, tables, zeros64)


def _tc_matmul(xp, W1):
    def body(x_ref, w_ref, o_ref):
        o_ref[...] = jnp.dot(x_ref[...], w_ref[...],
                             preferred_element_type=jnp.float32)

    return pl.pallas_call(
        body, out_shape=jax.ShapeDtypeStruct((NP, D), jnp.float32))(xp, W1)


def _tc_scale(h1, degs):
    def body(h_ref, d_ref, g_ref, s_ref):
        deg = d_ref[:NP, 0:1] + d_ref[NP:, 0:1] + 1.0
        dinv = lax.rsqrt(deg)
        g = h_ref[...] * dinv
        g_ref[:NP, :] = g[:, :DH]
        g_ref[NP:, :] = g[:, DH:]
        s_ref[...] = jnp.broadcast_to(dinv, (NP, D))

    return pl.pallas_call(
        body,
        out_shape=[jax.ShapeDtypeStruct((NC * NP, DH), jnp.float32),
                   jax.ShapeDtypeStruct((NP, D), jnp.float32)],
    )(h1, degs)


def _tc_layer(Pcols, g1cols, dinvb, b1, W2):
    def body(p_ref, g_ref, s_ref, b_ref, w_ref, o_ref):
        dinv = s_ref[...]
        acc = jnp.concatenate(
            [p_ref[:NP, :] + g_ref[:NP, :], p_ref[NP:, :] + g_ref[NP:, :]],
            axis=1)
        h = jnp.maximum(acc * dinv + b_ref[...], 0.0)
        g2 = jnp.dot(h, w_ref[...],
                     preferred_element_type=jnp.float32) * dinv
        o_ref[:NP, :] = g2[:, :DH]
        o_ref[NP:, :] = g2[:, DH:]

    return pl.pallas_call(
        body, out_shape=jax.ShapeDtypeStruct((NC * NP, DH), jnp.float32),
    )(Pcols, g1cols, dinvb, b1, W2)


def _tc_head(Qcols, g2cols, dinvb, b2, batch_p, fc_w, fc_b):
    def body(q_ref, g_ref, s_ref, b_ref, bt_ref, w_ref, c_ref, o_ref):
        acc = jnp.concatenate(
            [q_ref[:NP, :] + g_ref[:NP, :], q_ref[NP:, :] + g_ref[NP:, :]],
            axis=1)
        h = jnp.maximum(acc * s_ref[...] + b_ref[...], 0.0)
        gid = bt_ref[...]
        oh = (gid[:, None] == lax.broadcasted_iota(
            jnp.int32, (NP, NUM_GRAPHS), 1)).astype(jnp.float32)
        sums = lax.dot_general(oh, h, (((0,), (0,)), ((), ())),
                               preferred_element_type=jnp.float32)
        counts = jnp.sum(oh, axis=0)
        pooled = sums / jnp.maximum(counts, 1.0)[:, None]
        o_ref[...] = jnp.dot(pooled, w_ref[...],
                             preferred_element_type=jnp.float32) + c_ref[...]

    return pl.pallas_call(
        body,
        out_shape=jax.ShapeDtypeStruct((NUM_GRAPHS, fc_w.shape[1]), jnp.float32),
    )(Qcols, g2cols, dinvb, b2, batch_p, fc_w, fc_b)


def kernel(x, edge_index, batch, W1, b1, W2, b2, fc_w, fc_b):
    i32 = jnp.int32
    n = x.shape[0]
    src = edge_index[0].astype(i32)
    dst = edge_index[1].astype(i32)
    pad_e = EP - src.shape[0]
    src_f = jnp.concatenate([src, jnp.zeros((pad_e,), i32)])
    dst_f = jnp.concatenate([dst, jnp.full((pad_e,), GARBAGE, i32)])
    dst_deg = dst_f.reshape(NC, NS, NCH, CH)
    src_prop = jnp.stack([src_f, src_f + NP]).reshape(NC, NS, NCH_P, CH)
    dst_prop = dst_f.reshape(NS, NCH_P, CH)
    xp = jnp.pad(x, ((0, NP - n), (0, 0)))
    batch_p = jnp.concatenate([batch.astype(i32), jnp.full((NP - n,), NUM_GRAPHS, i32)])
    zeros16 = jnp.zeros((NP, 16), jnp.float32)
    ones16 = jnp.ones((CH, 16), jnp.float32)
    zeros64 = jnp.zeros((NP, DH), jnp.float32)

    degs = _sc_degree(dst_deg, zeros16, ones16)
    h1 = _tc_matmul(xp, W1)
    g1cols, dinvb = _tc_scale(h1, degs)
    P = _sc_propagate(src_prop, dst_prop, g1cols, zeros64)
    g2cols = _tc_layer(P, g1cols, dinvb, b1, W2)
    Q = _sc_propagate(src_prop, dst_prop, g2cols, zeros64)
    return _tc_head(Q, g2cols, dinvb, b2, batch_p, fc_w, fc_b)
